# baseline (speedup 1.0000x reference)
import jax
import jax.numpy as jnp
from jax import lax
from jax.experimental import pallas as pl
from jax.experimental.pallas import tpu as pltpu
from jax.experimental.pallas import tpu_sc as plsc

N = 10000
E = 320000
D = 128
NC = 2
NS = 16
NW = NC * NS
CHUNK = 64
NB = 4
NPH = 4
PH = 40
NCHUNK = NPH * PH
EPT = NCHUNK * CHUNK
EPP = PH * CHUNK
EPAD = NW * EPT
NP = 10240
RPT = NP // NS


def _sc_body(x_hbm, src_hbm, dst_hbm, w_hbm, agg_hbm,
             src_v, dst_v, w_v, b0, b1, b2, b3, agg_spmem,
             g0, g1, g2, g3, s0, s1, s2, s3, zsem):
    bufs = (b0, b1, b2, b3)
    gsems = (g0, g1, g2, g3)
    ssems = (s0, s1, s2, s3)
    cid = lax.axis_index("c")
    sid = lax.axis_index("s")
    wid = cid * NS + sid

    zero16 = jnp.zeros((16,), jnp.float32)

    def zrow(r, _):
        for j in range(D // 16):
            b0[r, pl.ds(j * 16, 16)] = zero16
        return 0

    lax.fori_loop(0, CHUNK, zrow, 0)
    for q in range(RPT // CHUNK):
        pltpu.async_copy(b0, agg_spmem.at[pl.ds(sid * RPT + q * CHUNK, CHUNK)],
                         zsem)
    for q in range(RPT // CHUNK):
        pltpu.make_async_copy(
            b0, agg_spmem.at[pl.ds(sid * RPT + q * CHUNK, CHUNK)], zsem).wait()
    plsc.subcore_barrier()

    def start_gather(c, k):
        pltpu.async_copy(x_hbm.at[src_v.at[pl.ds(c * CHUNK, CHUNK)]],
                         bufs[k], gsems[k])

    def wait_gather(k):
        pltpu.make_async_copy(x_hbm.at[src_v.at[pl.ds(0, CHUNK)]],
                              bufs[k], gsems[k]).wait()

    def start_scatter(c, k):
        pltpu.async_copy(bufs[k], agg_spmem.at[dst_v.at[c]], ssems[k],
                         add=True)

    def wait_scatter(k):
        pltpu.make_async_copy(bufs[k], agg_spmem.at[dst_v.at[0]],
                              ssems[k]).wait()

    def scale(c, buf):
        base = c * CHUNK

        def gbody(g, _):
            w16 = w_v[pl.ds(pl.multiple_of(base + g * 16, 16), 16)]
            for lane in range(16):
                we = jnp.full((16,), w16[lane], jnp.float32)
                row = g * 16 + lane
                for j in range(D // 16):
                    sl = pl.ds(j * 16, 16)
                    buf[row, sl] = buf[row, sl] * we
            return 0

        lax.fori_loop(0, CHUNK // 16, gbody, 0)

    def phase(p, _):
        pltpu.sync_copy(src_hbm.at[wid, pl.ds(p * EPP, EPP)], src_v)
        pltpu.sync_copy(dst_hbm.at[wid, pl.ds(p * PH, PH)], dst_v)
        pltpu.sync_copy(w_hbm.at[wid, pl.ds(p * EPP, EPP)], w_v)

        start_gather(0, 0)
        start_gather(1, 1)
        for c in range(2):
            wait_gather(c)
            start_gather(c + 2, c + 2)
            scale(c, bufs[c])
            start_scatter(c, c)

        def step(ii, _):
            for k4 in range(NB):
                k = (2 + k4) % NB
                c = 2 + ii * NB + k4
                wait_gather(k)
                wait_scatter((k + 2) % NB)
                start_gather(c + 2, (k + 2) % NB)
                scale(c, bufs[k])
                start_scatter(c, k)
            return 0

        lax.fori_loop(0, (PH - NB) // NB, step, 0)

        for c in range(PH - 2, PH):
            k = c % NB
            wait_gather(k)
            scale(c, bufs[k])
            start_scatter(c, k)

        for kk in range(NB):
            wait_scatter(kk)
        return 0

    lax.fori_loop(0, NPH, phase, 0)

    plsc.subcore_barrier()
    pltpu.sync_copy(agg_spmem.at[pl.ds(sid * RPT, RPT)],
                    agg_hbm.at[cid, pl.ds(sid * RPT, RPT)])


_sc_call = pl.kernel(
    _sc_body,
    out_type=jax.ShapeDtypeStruct((NC, NP, D), jnp.float32),
    mesh=plsc.VectorSubcoreMesh(core_axis_name="c", subcore_axis_name="s",
                                num_cores=NC, num_subcores=NS),
    scratch_types=[
        pltpu.VMEM((EPP,), jnp.int32),
        pltpu.VMEM((PH, CHUNK), jnp.int32),
        pltpu.VMEM((EPP,), jnp.float32),
        pltpu.VMEM((CHUNK, D), jnp.float32),
        pltpu.VMEM((CHUNK, D), jnp.float32),
        pltpu.VMEM((CHUNK, D), jnp.float32),
        pltpu.VMEM((CHUNK, D), jnp.float32),
        pltpu.VMEM_SHARED((NP, D), jnp.float32),
        pltpu.SemaphoreType.DMA,
        pltpu.SemaphoreType.DMA,
        pltpu.SemaphoreType.DMA,
        pltpu.SemaphoreType.DMA,
        pltpu.SemaphoreType.DMA,
        pltpu.SemaphoreType.DMA,
        pltpu.SemaphoreType.DMA,
        pltpu.SemaphoreType.DMA,
        pltpu.SemaphoreType.DMA,
    ],
)


def _mlp_body(x_ref, agg_ref, w1_ref, w2_ref, b_ref, o_ref):
    acc = jnp.dot(x_ref[...], w1_ref[...], preferred_element_type=jnp.float32)
    acc = acc + jnp.dot(agg_ref[0] + agg_ref[1], w2_ref[...],
                        preferred_element_type=jnp.float32)
    o_ref[...] = jnp.maximum(acc + b_ref[...], 0.0)


def kernel(x, edge_index, edge_weight, W, b):
    src = edge_index[0].astype(jnp.int32)
    dst = edge_index[1].astype(jnp.int32)
    w = edge_weight.astype(jnp.float32)

    pad = EPAD - E
    fill = (jnp.arange(pad, dtype=jnp.int32) * 97) % N
    src_p = jnp.concatenate([src, fill]).reshape(NW, EPT)
    dst_p = jnp.concatenate([dst, fill]).reshape(NW, NCHUNK, CHUNK)
    w_p = jnp.concatenate(
        [w, jnp.zeros((pad,), jnp.float32)]).reshape(NW, EPT)

    agg = _sc_call(x, src_p, dst_p, w_p)

    w1 = W[:D]
    w2 = W[D:]
    b2 = b.reshape(1, D)
    rows_blk = 1000
    out = pl.pallas_call(
        _mlp_body,
        grid=(N // rows_blk,),
        in_specs=[
            pl.BlockSpec((rows_blk, D), lambda i: (i, 0)),
            pl.BlockSpec((NC, rows_blk, D), lambda i: (0, i, 0)),
            pl.BlockSpec((D, D), lambda i: (0, 0)),
            pl.BlockSpec((D, D), lambda i: (0, 0)),
            pl.BlockSpec((1, D), lambda i: (0, 0)),
        ],
        out_specs=pl.BlockSpec((rows_blk, D), lambda i: (i, 0)),
        out_shape=jax.ShapeDtypeStruct((N, D), jnp.float32),
    )(x, agg, w1, w2, b2)
    return out
